# baseline (speedup 1.0000x reference)
import functools

import jax
import jax.numpy as jnp
from jax import lax
from jax.experimental import pallas as pl
from jax.experimental.pallas import tpu as pltpu
from jax.experimental.pallas import tpu_sc as plsc

B = 8
N = 8192
S = 1024
C = 128
CHUNK = 128


NBLK = N // 128
GROUP = 16


def _fps_body(px_ref, py_ref, pz_ref, px3_ref, py3_ref, pz3_ref,
              idx_ref, cx_ref, cy_ref, cz_ref, dist_ref):
    lane = lax.broadcasted_iota(jnp.int32, (B, 128), 1)
    rowoff = lax.broadcasted_iota(jnp.int32, (B, 1), 0) * N

    def sweep(cx, cy, cz, init_store):
        zf = jnp.zeros((B, 128), jnp.float32)
        zi = jnp.zeros((B, 128), jnp.int32)
        acc0 = (jnp.full((B, 128), -1.0, jnp.float32), zi, zf, zf, zf)
        accs = (acc0, acc0)

        def grp(gi, accs):
            accs = list(accs)
            for u in range(GROUP):
                k = gi * GROUP + u
                Xk = px3_ref[k]
                Yk = py3_ref[k]
                Zk = pz3_ref[k]
                d = (Xk - cx) ** 2 + (Yk - cy) ** 2 + (Zk - cz) ** 2
                if init_store:
                    dist_ref[k] = jnp.full((B, 128), 1e10, jnp.float32)
                    nd = d
                else:
                    nd = jnp.minimum(dist_ref[k], d)
                    dist_ref[k] = nd
                gc = lane + k * 128
                av, ag, ax, ay, az = accs[u & 1]
                m = av >= nd
                accs[u & 1] = (
                    jnp.where(m, av, nd),
                    jnp.where(m, ag, gc),
                    jnp.where(m, ax, Xk),
                    jnp.where(m, ay, Yk),
                    jnp.where(m, az, Zk),
                )
            return tuple(accs)

        accs = lax.fori_loop(0, NBLK // GROUP, grp, accs)
        (av1, ag1, ax1, ay1, az1), (av2, ag2, ax2, ay2, az2) = accs
        m12 = (av1 > av2) | ((av1 == av2) & (ag1 < ag2))
        av = jnp.where(m12, av1, av2)
        ag = jnp.where(m12, ag1, ag2)
        ax = jnp.where(m12, ax1, ax2)
        ay = jnp.where(m12, ay1, ay2)
        az = jnp.where(m12, az1, az2)
        for s in (1, 2, 4, 8, 16, 32, 64):
            rv = pltpu.roll(av, s, 1)
            rg = pltpu.roll(ag, s, 1)
            rx = pltpu.roll(ax, s, 1)
            ry = pltpu.roll(ay, s, 1)
            rz = pltpu.roll(az, s, 1)
            m = (av > rv) | ((av == rv) & (ag < rg))
            av = jnp.where(m, av, rv)
            ag = jnp.where(m, ag, rg)
            ax = jnp.where(m, ax, rx)
            ay = jnp.where(m, ay, ry)
            az = jnp.where(m, az, rz)
        return ag, ax, ay, az

    X = px_ref[...]
    Y = py_ref[...]
    Z = pz_ref[...]
    n = jnp.float32(N)
    mx = jnp.sum(X, axis=1, keepdims=True) / n
    my = jnp.sum(Y, axis=1, keepdims=True) / n
    mz = jnp.sum(Z, axis=1, keepdims=True) / n
    far, cx, cy, cz = sweep(mx, my, mz, True)

    def body(j, carry):
        far, cx, cy, cz, ia, xa, ya, za = carry
        sel = lane == j
        ia = jnp.where(sel, far + rowoff, ia)
        xa = jnp.where(sel, cx, xa)
        ya = jnp.where(sel, cy, ya)
        za = jnp.where(sel, cz, za)
        far, cx, cy, cz = sweep(cx, cy, cz, False)
        return far, cx, cy, cz, ia, xa, ya, za

    zf = jnp.zeros((B, CHUNK), jnp.float32)
    zi = jnp.zeros((B, CHUNK), jnp.int32)
    for c in range(S // CHUNK):
        far, cx, cy, cz, ia, xa, ya, za = lax.fori_loop(
            0, CHUNK, body, (far, cx, cy, cz, zi, zf, zf, zf)
        )
        sl = slice(c * CHUNK, (c + 1) * CHUNK)
        idx_ref[:, sl] = ia
        cx_ref[:, sl] = xa
        cy_ref[:, sl] = ya
        cz_ref[:, sl] = za


_fps = pl.pallas_call(
    _fps_body,
    out_shape=[
        jax.ShapeDtypeStruct((B, S), jnp.int32),
        jax.ShapeDtypeStruct((B, S), jnp.float32),
        jax.ShapeDtypeStruct((B, S), jnp.float32),
        jax.ShapeDtypeStruct((B, S), jnp.float32),
    ],
    scratch_shapes=[pltpu.VMEM((NBLK, B, 128), jnp.float32)],
)


def _make_gather():
    info = plsc.get_sparse_core_info()
    nw = info.num_cores * info.num_subcores
    per = (B * S) // nw
    mesh = plsc.VectorSubcoreMesh(core_axis_name="c", subcore_axis_name="s")

    @functools.partial(
        pl.kernel,
        mesh=mesh,
        out_type=jax.ShapeDtypeStruct((B * S, C), jnp.float32),
        scratch_types=[
            pltpu.VMEM((per,), jnp.int32),
            pltpu.VMEM((per, C), jnp.float32),
            pltpu.SemaphoreType.DMA,
        ],
    )
    def gather_k(table_hbm, idx_hbm, out_hbm, idx_v, rows_v, sem):
        wid = lax.axis_index("s") * info.num_cores + lax.axis_index("c")
        base = wid * per
        pltpu.sync_copy(idx_hbm.at[pl.ds(base, per)], idx_v)
        pltpu.async_copy(table_hbm.at[idx_v], rows_v, sem).wait()
        pltpu.sync_copy(rows_v, out_hbm.at[pl.ds(base, per)])

    return gather_k


@jax.jit
def kernel(points, features):
    px = points[:, 0, :]
    py = points[:, 1, :]
    pz = points[:, 2, :]
    px3 = px.reshape(B, NBLK, 128).swapaxes(0, 1)
    py3 = py.reshape(B, NBLK, 128).swapaxes(0, 1)
    pz3 = pz.reshape(B, NBLK, 128).swapaxes(0, 1)
    gidx, cxo, cyo, czo = _fps(px, py, pz, px3, py3, pz3)
    sampled_points = jnp.stack([cxo, cyo, czo], axis=-1)
    table = jnp.swapaxes(features, -1, -2).reshape(B * N, C)
    flat = _make_gather()(table, gidx.reshape(B * S))
    sampled_features = flat.reshape(B, S, C)
    return sampled_points, sampled_features


# speedup vs baseline: 1.5935x; 1.5935x over previous
import functools

import jax
import jax.numpy as jnp
from jax import lax
from jax.experimental import pallas as pl
from jax.experimental.pallas import tpu as pltpu
from jax.experimental.pallas import tpu_sc as plsc

B = 8
N = 8192
S = 1024
C = 128
CHUNK = 128


NBLK = N // 128
GROUP = 16


def _fps_body(px_ref, py_ref, pz_ref, px3_ref, py3_ref, pz3_ref,
              idx_ref, cx_ref, cy_ref, cz_ref, dist_ref):
    lane = lax.broadcasted_iota(jnp.int32, (B, 128), 1)
    lanef = lane.astype(jnp.float32)
    rowoff = lax.broadcasted_iota(jnp.int32, (B, 1), 0) * N

    def sweep(cx, cy, cz, init_store):
        zf = jnp.zeros((B, 128), jnp.float32)
        acc0 = (jnp.full((B, 128), -1.0, jnp.float32), zf, zf, zf, zf)
        accs = (acc0, acc0)

        def grp(gi, accs):
            accs = list(accs)
            for u in range(GROUP):
                k = gi * GROUP + u
                Xk = px3_ref[k]
                Yk = py3_ref[k]
                Zk = pz3_ref[k]
                d = (Xk - cx) ** 2 + (Yk - cy) ** 2 + (Zk - cz) ** 2
                if init_store:
                    dist_ref[k] = jnp.full((B, 128), 1e10, jnp.float32)
                    nd = d
                else:
                    nd = jnp.minimum(dist_ref[k], d)
                    dist_ref[k] = nd
                gc = lanef + k * 128.0
                av, ag, ax, ay, az = accs[u & 1]
                m = av >= nd
                accs[u & 1] = (
                    jnp.where(m, av, nd),
                    jnp.where(m, ag, gc),
                    jnp.where(m, ax, Xk),
                    jnp.where(m, ay, Yk),
                    jnp.where(m, az, Zk),
                )
            return tuple(accs)

        accs = lax.fori_loop(0, NBLK // GROUP, grp, accs)
        (av1, ag1, ax1, ay1, az1), (av2, ag2, ax2, ay2, az2) = accs
        m12 = (av1 > av2) | ((av1 == av2) & (ag1 < ag2))
        av = jnp.where(m12, av1, av2)
        ag = jnp.where(m12, ag1, ag2)
        ax = jnp.where(m12, ax1, ax2)
        ay = jnp.where(m12, ay1, ay2)
        az = jnp.where(m12, az1, az2)
        mrow = jnp.max(av, axis=1, keepdims=True)
        gc = jnp.where(av == mrow, ag, 8192.0)
        g = jnp.min(gc, axis=1, keepdims=True)
        oh = gc == g
        ncx = jnp.sum(jnp.where(oh, ax, zf), axis=1, keepdims=True)
        ncy = jnp.sum(jnp.where(oh, ay, zf), axis=1, keepdims=True)
        ncz = jnp.sum(jnp.where(oh, az, zf), axis=1, keepdims=True)
        return g, ncx, ncy, ncz

    X = px_ref[...]
    Y = py_ref[...]
    Z = pz_ref[...]
    n = jnp.float32(N)
    mx = jnp.sum(X, axis=1, keepdims=True) / n
    my = jnp.sum(Y, axis=1, keepdims=True) / n
    mz = jnp.sum(Z, axis=1, keepdims=True) / n
    far, cx, cy, cz = sweep(mx, my, mz, True)

    def body(j, carry):
        far, cx, cy, cz, ia, xa, ya, za = carry
        sel = lane == j
        ia = jnp.where(sel, far, ia)
        xa = jnp.where(sel, cx, xa)
        ya = jnp.where(sel, cy, ya)
        za = jnp.where(sel, cz, za)
        far, cx, cy, cz = sweep(cx, cy, cz, False)
        return far, cx, cy, cz, ia, xa, ya, za

    zf = jnp.zeros((B, CHUNK), jnp.float32)
    for c in range(S // CHUNK):
        far, cx, cy, cz, ia, xa, ya, za = lax.fori_loop(
            0, CHUNK, body, (far, cx, cy, cz, zf, zf, zf, zf)
        )
        sl = slice(c * CHUNK, (c + 1) * CHUNK)
        idx_ref[:, sl] = ia.astype(jnp.int32) + rowoff
        cx_ref[:, sl] = xa
        cy_ref[:, sl] = ya
        cz_ref[:, sl] = za


_fps = pl.pallas_call(
    _fps_body,
    out_shape=[
        jax.ShapeDtypeStruct((B, S), jnp.int32),
        jax.ShapeDtypeStruct((B, S), jnp.float32),
        jax.ShapeDtypeStruct((B, S), jnp.float32),
        jax.ShapeDtypeStruct((B, S), jnp.float32),
    ],
    scratch_shapes=[pltpu.VMEM((NBLK, B, 128), jnp.float32)],
)


def _make_gather():
    info = plsc.get_sparse_core_info()
    nw = info.num_cores * info.num_subcores
    per = (B * S) // nw
    mesh = plsc.VectorSubcoreMesh(core_axis_name="c", subcore_axis_name="s")

    @functools.partial(
        pl.kernel,
        mesh=mesh,
        out_type=jax.ShapeDtypeStruct((B * S, C), jnp.float32),
        scratch_types=[
            pltpu.VMEM((per,), jnp.int32),
            pltpu.VMEM((per, C), jnp.float32),
            pltpu.SemaphoreType.DMA,
        ],
    )
    def gather_k(table_hbm, idx_hbm, out_hbm, idx_v, rows_v, sem):
        wid = lax.axis_index("s") * info.num_cores + lax.axis_index("c")
        base = wid * per
        pltpu.sync_copy(idx_hbm.at[pl.ds(base, per)], idx_v)
        pltpu.async_copy(table_hbm.at[idx_v], rows_v, sem).wait()
        pltpu.sync_copy(rows_v, out_hbm.at[pl.ds(base, per)])

    return gather_k


@jax.jit
def kernel(points, features):
    px = points[:, 0, :]
    py = points[:, 1, :]
    pz = points[:, 2, :]
    px3 = px.reshape(B, NBLK, 128).swapaxes(0, 1)
    py3 = py.reshape(B, NBLK, 128).swapaxes(0, 1)
    pz3 = pz.reshape(B, NBLK, 128).swapaxes(0, 1)
    gidx, cxo, cyo, czo = _fps(px, py, pz, px3, py3, pz3)
    sampled_points = jnp.stack([cxo, cyo, czo], axis=-1)
    table = jnp.swapaxes(features, -1, -2).reshape(B * N, C)
    flat = _make_gather()(table, gidx.reshape(B * S))
    sampled_features = flat.reshape(B, S, C)
    return sampled_points, sampled_features


# speedup vs baseline: 1.6091x; 1.0098x over previous
import functools

import jax
import jax.numpy as jnp
from jax import lax
from jax.experimental import pallas as pl
from jax.experimental.pallas import tpu as pltpu
from jax.experimental.pallas import tpu_sc as plsc

B = 8
N = 8192
S = 1024
C = 128
CHUNK = 128


NBLK = N // 128
GROUP = 32


def _fps_body(px_ref, py_ref, pz_ref, px3_ref, py3_ref, pz3_ref,
              idx_ref, cx_ref, cy_ref, cz_ref, dist_ref):
    lane = lax.broadcasted_iota(jnp.int32, (B, 128), 1)
    lanef = lane.astype(jnp.float32)
    rowoff = lax.broadcasted_iota(jnp.int32, (B, 1), 0) * N

    def sweep(cx, cy, cz, init_store):
        zf = jnp.zeros((B, 128), jnp.float32)
        acc0 = (jnp.full((B, 128), -1.0, jnp.float32), zf, zf, zf, zf)
        accs = (acc0, acc0)

        def grp(gi, accs):
            accs = list(accs)
            for u in range(GROUP):
                k = gi * GROUP + u
                Xk = px3_ref[k]
                Yk = py3_ref[k]
                Zk = pz3_ref[k]
                d = (Xk - cx) ** 2 + (Yk - cy) ** 2 + (Zk - cz) ** 2
                if init_store:
                    dist_ref[k] = jnp.full((B, 128), 1e10, jnp.float32)
                    nd = d
                else:
                    nd = jnp.minimum(dist_ref[k], d)
                    dist_ref[k] = nd
                gc = lanef + k * 128.0
                av, ag, ax, ay, az = accs[u & 1]
                m = av >= nd
                accs[u & 1] = (
                    jnp.where(m, av, nd),
                    jnp.where(m, ag, gc),
                    jnp.where(m, ax, Xk),
                    jnp.where(m, ay, Yk),
                    jnp.where(m, az, Zk),
                )
            return tuple(accs)

        accs = lax.fori_loop(0, NBLK // GROUP, grp, accs)
        (av1, ag1, ax1, ay1, az1), (av2, ag2, ax2, ay2, az2) = accs
        m12 = (av1 > av2) | ((av1 == av2) & (ag1 < ag2))
        av = jnp.where(m12, av1, av2)
        ag = jnp.where(m12, ag1, ag2)
        ax = jnp.where(m12, ax1, ax2)
        ay = jnp.where(m12, ay1, ay2)
        az = jnp.where(m12, az1, az2)
        mrow = jnp.max(av, axis=1, keepdims=True)
        gc = jnp.where(av == mrow, ag, 8192.0)
        g = jnp.min(gc, axis=1, keepdims=True)
        oh = gc == g
        ncx = jnp.sum(jnp.where(oh, ax, zf), axis=1, keepdims=True)
        ncy = jnp.sum(jnp.where(oh, ay, zf), axis=1, keepdims=True)
        ncz = jnp.sum(jnp.where(oh, az, zf), axis=1, keepdims=True)
        return g, ncx, ncy, ncz

    X = px_ref[...]
    Y = py_ref[...]
    Z = pz_ref[...]
    n = jnp.float32(N)
    mx = jnp.sum(X, axis=1, keepdims=True) / n
    my = jnp.sum(Y, axis=1, keepdims=True) / n
    mz = jnp.sum(Z, axis=1, keepdims=True) / n
    far, cx, cy, cz = sweep(mx, my, mz, True)

    def body(j, carry):
        far, cx, cy, cz, ia, xa, ya, za = carry
        sel = lane == j
        ia = jnp.where(sel, far, ia)
        xa = jnp.where(sel, cx, xa)
        ya = jnp.where(sel, cy, ya)
        za = jnp.where(sel, cz, za)
        far, cx, cy, cz = sweep(cx, cy, cz, False)
        return far, cx, cy, cz, ia, xa, ya, za

    zf = jnp.zeros((B, CHUNK), jnp.float32)
    for c in range(S // CHUNK):
        far, cx, cy, cz, ia, xa, ya, za = lax.fori_loop(
            0, CHUNK, body, (far, cx, cy, cz, zf, zf, zf, zf)
        )
        sl = slice(c * CHUNK, (c + 1) * CHUNK)
        idx_ref[:, sl] = ia.astype(jnp.int32) + rowoff
        cx_ref[:, sl] = xa
        cy_ref[:, sl] = ya
        cz_ref[:, sl] = za


_fps = pl.pallas_call(
    _fps_body,
    out_shape=[
        jax.ShapeDtypeStruct((B, S), jnp.int32),
        jax.ShapeDtypeStruct((B, S), jnp.float32),
        jax.ShapeDtypeStruct((B, S), jnp.float32),
        jax.ShapeDtypeStruct((B, S), jnp.float32),
    ],
    scratch_shapes=[pltpu.VMEM((NBLK, B, 128), jnp.float32)],
)


def _make_gather():
    info = plsc.get_sparse_core_info()
    nw = info.num_cores * info.num_subcores
    per = (B * S) // nw
    mesh = plsc.VectorSubcoreMesh(core_axis_name="c", subcore_axis_name="s")

    @functools.partial(
        pl.kernel,
        mesh=mesh,
        out_type=jax.ShapeDtypeStruct((B * S, C), jnp.float32),
        scratch_types=[
            pltpu.VMEM((per,), jnp.int32),
            pltpu.VMEM((per, C), jnp.float32),
            pltpu.SemaphoreType.DMA,
        ],
    )
    def gather_k(table_hbm, idx_hbm, out_hbm, idx_v, rows_v, sem):
        wid = lax.axis_index("s") * info.num_cores + lax.axis_index("c")
        base = wid * per
        pltpu.sync_copy(idx_hbm.at[pl.ds(base, per)], idx_v)
        pltpu.async_copy(table_hbm.at[idx_v], rows_v, sem).wait()
        pltpu.sync_copy(rows_v, out_hbm.at[pl.ds(base, per)])

    return gather_k


@jax.jit
def kernel(points, features):
    px = points[:, 0, :]
    py = points[:, 1, :]
    pz = points[:, 2, :]
    px3 = px.reshape(B, NBLK, 128).swapaxes(0, 1)
    py3 = py.reshape(B, NBLK, 128).swapaxes(0, 1)
    pz3 = pz.reshape(B, NBLK, 128).swapaxes(0, 1)
    gidx, cxo, cyo, czo = _fps(px, py, pz, px3, py3, pz3)
    sampled_points = jnp.stack([cxo, cyo, czo], axis=-1)
    table = jnp.swapaxes(features, -1, -2).reshape(B * N, C)
    flat = _make_gather()(table, gidx.reshape(B * S))
    sampled_features = flat.reshape(B, S, C)
    return sampled_points, sampled_features


# speedup vs baseline: 1.7370x; 1.0795x over previous
import functools

import jax
import jax.numpy as jnp
from jax import lax
from jax.experimental import pallas as pl
from jax.experimental.pallas import tpu as pltpu
from jax.experimental.pallas import tpu_sc as plsc

B = 8
N = 8192
S = 1024
C = 128
CHUNK = 128


NBLK = N // 128
GROUP = 64


def _fps_body(px_ref, py_ref, pz_ref, px3_ref, py3_ref, pz3_ref,
              idx_ref, cx_ref, cy_ref, cz_ref, dist_ref):
    lane = lax.broadcasted_iota(jnp.int32, (B, 128), 1)
    lanef = lane.astype(jnp.float32)
    rowoff = lax.broadcasted_iota(jnp.int32, (B, 1), 0) * N

    def sweep(cx, cy, cz, init_store):
        zf = jnp.zeros((B, 128), jnp.float32)
        acc0 = (jnp.full((B, 128), -1.0, jnp.float32), zf, zf, zf, zf)
        accs = (acc0, acc0)

        def grp(gi, accs):
            accs = list(accs)
            for u in range(GROUP):
                k = gi * GROUP + u
                Xk = px3_ref[k]
                Yk = py3_ref[k]
                Zk = pz3_ref[k]
                d = (Xk - cx) ** 2 + (Yk - cy) ** 2 + (Zk - cz) ** 2
                if init_store:
                    dist_ref[k] = jnp.full((B, 128), 1e10, jnp.float32)
                    nd = d
                else:
                    nd = jnp.minimum(dist_ref[k], d)
                    dist_ref[k] = nd
                gc = lanef + k * 128.0
                av, ag, ax, ay, az = accs[u & 1]
                m = av >= nd
                accs[u & 1] = (
                    jnp.where(m, av, nd),
                    jnp.where(m, ag, gc),
                    jnp.where(m, ax, Xk),
                    jnp.where(m, ay, Yk),
                    jnp.where(m, az, Zk),
                )
            return tuple(accs)

        if NBLK // GROUP == 1:
            accs = grp(0, accs)
        else:
            accs = lax.fori_loop(0, NBLK // GROUP, grp, accs)
        (av1, ag1, ax1, ay1, az1), (av2, ag2, ax2, ay2, az2) = accs
        m12 = (av1 > av2) | ((av1 == av2) & (ag1 < ag2))
        av = jnp.where(m12, av1, av2)
        ag = jnp.where(m12, ag1, ag2)
        ax = jnp.where(m12, ax1, ax2)
        ay = jnp.where(m12, ay1, ay2)
        az = jnp.where(m12, az1, az2)
        mrow = jnp.max(av, axis=1, keepdims=True)
        gc = jnp.where(av == mrow, ag, 8192.0)
        g = jnp.min(gc, axis=1, keepdims=True)
        oh = gc == g
        ncx = jnp.sum(jnp.where(oh, ax, zf), axis=1, keepdims=True)
        ncy = jnp.sum(jnp.where(oh, ay, zf), axis=1, keepdims=True)
        ncz = jnp.sum(jnp.where(oh, az, zf), axis=1, keepdims=True)
        return g, ncx, ncy, ncz

    X = px_ref[...]
    Y = py_ref[...]
    Z = pz_ref[...]
    n = jnp.float32(N)
    mx = jnp.sum(X, axis=1, keepdims=True) / n
    my = jnp.sum(Y, axis=1, keepdims=True) / n
    mz = jnp.sum(Z, axis=1, keepdims=True) / n
    far, cx, cy, cz = sweep(mx, my, mz, True)

    def body(j, carry):
        far, cx, cy, cz, ia, xa, ya, za = carry
        sel = lane == j
        ia = jnp.where(sel, far, ia)
        xa = jnp.where(sel, cx, xa)
        ya = jnp.where(sel, cy, ya)
        za = jnp.where(sel, cz, za)
        far, cx, cy, cz = sweep(cx, cy, cz, False)
        return far, cx, cy, cz, ia, xa, ya, za

    zf = jnp.zeros((B, CHUNK), jnp.float32)
    for c in range(S // CHUNK):
        far, cx, cy, cz, ia, xa, ya, za = lax.fori_loop(
            0, CHUNK, body, (far, cx, cy, cz, zf, zf, zf, zf)
        )
        sl = slice(c * CHUNK, (c + 1) * CHUNK)
        idx_ref[:, sl] = ia.astype(jnp.int32) + rowoff
        cx_ref[:, sl] = xa
        cy_ref[:, sl] = ya
        cz_ref[:, sl] = za


_fps = pl.pallas_call(
    _fps_body,
    out_shape=[
        jax.ShapeDtypeStruct((B, S), jnp.int32),
        jax.ShapeDtypeStruct((B, S), jnp.float32),
        jax.ShapeDtypeStruct((B, S), jnp.float32),
        jax.ShapeDtypeStruct((B, S), jnp.float32),
    ],
    scratch_shapes=[pltpu.VMEM((NBLK, B, 128), jnp.float32)],
)


def _make_gather():
    info = plsc.get_sparse_core_info()
    nw = info.num_cores * info.num_subcores
    per = (B * S) // nw
    mesh = plsc.VectorSubcoreMesh(core_axis_name="c", subcore_axis_name="s")

    @functools.partial(
        pl.kernel,
        mesh=mesh,
        out_type=jax.ShapeDtypeStruct((B * S, C), jnp.float32),
        scratch_types=[
            pltpu.VMEM((per,), jnp.int32),
            pltpu.VMEM((per, C), jnp.float32),
            pltpu.SemaphoreType.DMA,
        ],
    )
    def gather_k(table_hbm, idx_hbm, out_hbm, idx_v, rows_v, sem):
        wid = lax.axis_index("s") * info.num_cores + lax.axis_index("c")
        base = wid * per
        pltpu.sync_copy(idx_hbm.at[pl.ds(base, per)], idx_v)
        pltpu.async_copy(table_hbm.at[idx_v], rows_v, sem).wait()
        pltpu.sync_copy(rows_v, out_hbm.at[pl.ds(base, per)])

    return gather_k


@jax.jit
def kernel(points, features):
    px = points[:, 0, :]
    py = points[:, 1, :]
    pz = points[:, 2, :]
    px3 = px.reshape(B, NBLK, 128).swapaxes(0, 1)
    py3 = py.reshape(B, NBLK, 128).swapaxes(0, 1)
    pz3 = pz.reshape(B, NBLK, 128).swapaxes(0, 1)
    gidx, cxo, cyo, czo = _fps(px, py, pz, px3, py3, pz3)
    sampled_points = jnp.stack([cxo, cyo, czo], axis=-1)
    table = jnp.swapaxes(features, -1, -2).reshape(B * N, C)
    flat = _make_gather()(table, gidx.reshape(B * S))
    sampled_features = flat.reshape(B, S, C)
    return sampled_points, sampled_features
